# baseline (speedup 1.0000x reference)
import functools

import jax
import jax.numpy as jnp
from jax import lax
from jax.experimental import pallas as pl
from jax.experimental.pallas import tpu as pltpu
from jax.experimental.pallas import tpu_sc as plsc

N = 10000
E = 320000
D = 128
H = 4
NEG = 0.2

NT = 16
EPT = E // NT
B = 32
NCH = EPT // B
NG = B // 16
NB = 10
BR = N // NB


def _prep_body(x_ref, wl_ref, wr_ref, xl_ref, xr_ref):
    x = x_ref[...]
    xl_ref[0] = jnp.dot(x, wl_ref[0], preferred_element_type=jnp.float32)
    xr_ref[0] = jnp.dot(x, wr_ref[0], preferred_element_type=jnp.float32)


def _prep(x, wl_h, wr_h):
    return pl.pallas_call(
        _prep_body,
        grid=(H, NB),
        in_specs=[
            pl.BlockSpec((BR, D), lambda h, i: (i, 0)),
            pl.BlockSpec((1, D, D), lambda h, i: (h, 0, 0)),
            pl.BlockSpec((1, D, D), lambda h, i: (h, 0, 0)),
        ],
        out_specs=[
            pl.BlockSpec((1, BR, D), lambda h, i: (h, i, 0)),
            pl.BlockSpec((1, BR, D), lambda h, i: (h, i, 0)),
        ],
        out_shape=[
            jax.ShapeDtypeStruct((H, N, D), jnp.float32),
            jax.ShapeDtypeStruct((H, N, D), jnp.float32),
        ],
    )(x, wl_h, wr_h)


def _sc_body(xl_hbm, xr_hbm, src_hbm, dst_hbm, att_hbm, out_hbm,
             srcb0, dstb0, sidx0, didx0, ul0, ur0, ivb0, w0,
             srcb1, dstb1, sidx1, didx1, ul1, ur1, ivb1, w1,
             ex_v, acc_v, dch_v, z640_v, att_v,
             denom_s, out_s,
             semi0a, semi0b, semr0a, semr0b,
             semi1a, semi1b, semr1a, semr1b):
    cid = lax.axis_index("c")
    sid = lax.axis_index("s")

    bufA = (srcb0, dstb0, sidx0, didx0, ul0, ur0, ivb0, w0,
            semi0a, semr0a, semi0b, semr0b)
    bufB = (srcb1, dstb1, sidx1, didx1, ul1, ur1, ivb1, w1,
            semi1a, semr1a, semi1b, semr1b)

    pltpu.sync_copy(att_hbm, att_v)

    zv = jnp.zeros((16,), jnp.float32)
    lane = lax.iota(jnp.int32, 16)
    laneb = lane * 16

    def _z640(i, c):
        z640_v[pl.ds(i * 16, 16)] = zv
        return c
    lax.fori_loop(0, 40, _z640, 0)

    ebase = sid * EPT

    def _fire_idx(j, s):
        off = ebase + j * B
        c1 = pltpu.async_copy(src_hbm.at[pl.ds(off, B)], s[0], s[8])
        c2 = pltpu.async_copy(dst_hbm.at[pl.ds(off, B)], s[1], s[10])
        return (c1, c2)

    def _wait(cs):
        for c in cs:
            c.wait()

    for hh in range(2):
        head = cid * 2 + hh
        base = head * N

        def _zul(r, c):
            for k in range(8):
                ul0[r, pl.ds(k * 16, 16)] = zv
            return c
        lax.fori_loop(0, B, _zul, 0)

        @pl.when(sid < 15)
        def _():
            pltpu.sync_copy(z640_v, denom_s.at[pl.ds(sid * 640, 640)])
            for k in range(20):
                pltpu.sync_copy(ul0, out_s.at[pl.ds(sid * 640 + k * B, B)])

        @pl.when(sid == 15)
        def _():
            pltpu.sync_copy(z640_v.at[pl.ds(0, 400)],
                            denom_s.at[pl.ds(9600, 400)])
            for k in range(12):
                pltpu.sync_copy(ul0, out_s.at[pl.ds(9600 + k * B, B)])
            pltpu.sync_copy(ul0.at[pl.ds(0, 16)], out_s.at[pl.ds(9984, 16)])
        plsc.subcore_barrier()

        att_c = [att_v[pl.ds(head * D + c * 16, 16)] for c in range(8)]

        def _p1_build(s):
            for k in range(NG):
                s[2][pl.ds(k * 16, 16)] = s[0][pl.ds(k * 16, 16)] + base
                s[3][pl.ds(k * 16, 16)] = s[1][pl.ds(k * 16, 16)] + base

        def _p1_fire_rows(s):
            c1 = pltpu.async_copy(xl_hbm.at[s[2]], s[4], s[9])
            c2 = pltpu.async_copy(xr_hbm.at[s[3]], s[5], s[11])
            return (c1, c2)

        def _p1_compute(j, s):
            ul, ur = s[4], s[5]
            toff = pl.multiple_of(j * B, B)

            @plsc.parallel_loop(0, B, unroll=4)
            def _edge(e):
                acc = zv
                for c3 in range(8):
                    t = ul[e, pl.ds(c3 * 16, 16)] + ur[e, pl.ds(c3 * 16, 16)]
                    t = jnp.maximum(t, NEG * t)
                    acc = acc + att_c[c3] * t
                acc_v[pl.ds(e * 16, 16)] = acc

            @plsc.parallel_loop(0, NG)
            def _red(k):
                tot = zv
                for jj in range(16):
                    tot = tot + plsc.load_gather(acc_v,
                                                 [laneb + (k * 256 + jj)])
                ex_v[pl.ds(toff + k * 16, 16)] = jnp.exp(tot)
            pltpu.sync_copy(ex_v.at[pl.ds(toff, B)],
                            denom_s.at[s[1]], add=True)

        def _p1_body(i, c):
            a = 2 * i
            b = 2 * i + 1
            ia = _fire_idx(a, bufA)
            ib = _fire_idx(b, bufB)
            _wait(ia)
            _p1_build(bufA)
            ra = _p1_fire_rows(bufA)
            _wait(ib)
            _p1_build(bufB)
            rb = _p1_fire_rows(bufB)
            _wait(ra)
            _p1_compute(a, bufA)
            _wait(rb)
            _p1_compute(b, bufB)
            return c
        lax.fori_loop(0, NCH // 2, _p1_body, 0)
        it = _fire_idx(NCH - 1, bufA)
        _wait(it)
        _p1_build(bufA)
        rt = _p1_fire_rows(bufA)
        _wait(rt)
        _p1_compute(NCH - 1, bufA)
        plsc.subcore_barrier()

        def _inv_stripe(n, r0):
            pltpu.sync_copy(denom_s.at[pl.ds(r0, n)], dch_v.at[pl.ds(0, n)])

            def _i(i, c):
                v = dch_v[pl.ds(i * 16, 16)]
                dch_v[pl.ds(i * 16, 16)] = 1.0 / (v + 1e-16)
                return c
            lax.fori_loop(0, n // 16, _i, 0)
            pltpu.sync_copy(dch_v.at[pl.ds(0, n)], denom_s.at[pl.ds(r0, n)])

        @pl.when(sid < 15)
        def _():
            _inv_stripe(640, sid * 640)

        @pl.when(sid == 15)
        def _():
            _inv_stripe(400, 9600)
        plsc.subcore_barrier()

        def _p2_build(s):
            for k in range(NG):
                s[2][pl.ds(k * 16, 16)] = s[0][pl.ds(k * 16, 16)] + base

        def _p2_fire_rows(s):
            c1 = pltpu.async_copy(xl_hbm.at[s[2]], s[4], s[9])
            c2 = pltpu.async_copy(denom_s.at[s[1]], s[6], s[11])
            return (c1, c2)

        def _p2_compute(j, s):
            ul, ivb, wv = s[4], s[6], s[7]
            toff = pl.multiple_of(j * B, B)
            for k in range(NG):
                wv[pl.ds(k * 16, 16)] = (ex_v[pl.ds(toff + k * 16, 16)]
                                         * ivb[pl.ds(k * 16, 16)])

            @plsc.parallel_loop(0, B, unroll=4)
            def _edge(e):
                w = plsc.load_gather(wv, [jnp.full((16,), e, jnp.int32)])
                for c2 in range(8):
                    ul[e, pl.ds(c2 * 16, 16)] = w * ul[e, pl.ds(c2 * 16, 16)]
            pltpu.sync_copy(ul, out_s.at[s[1]], add=True)

        def _p2_body(i, c):
            a = 2 * i
            b = 2 * i + 1
            ia = _fire_idx(a, bufA)
            ib = _fire_idx(b, bufB)
            _wait(ia)
            _p2_build(bufA)
            ra = _p2_fire_rows(bufA)
            _wait(ib)
            _p2_build(bufB)
            rb = _p2_fire_rows(bufB)
            _wait(ra)
            _p2_compute(a, bufA)
            _wait(rb)
            _p2_compute(b, bufB)
            return c
        lax.fori_loop(0, NCH // 2, _p2_body, 0)
        it = _fire_idx(NCH - 1, bufA)
        _wait(it)
        _p2_build(bufA)
        rt = _p2_fire_rows(bufA)
        _wait(rt)
        _p2_compute(NCH - 1, bufA)
        plsc.subcore_barrier()

        @pl.when(sid < 15)
        def _():
            pltpu.sync_copy(out_s.at[pl.ds(sid * 640, 640)],
                            out_hbm.at[pl.ds(base + sid * 640, 640)])

        @pl.when(sid == 15)
        def _():
            pltpu.sync_copy(out_s.at[pl.ds(9600, 400)],
                            out_hbm.at[pl.ds(base + 9600, 400)])


def _sc_edges(xl_t, xr_t, src_r, dst_r, att_f):
    mesh = plsc.VectorSubcoreMesh(core_axis_name="c", subcore_axis_name="s")
    set_bufs = [
        pltpu.VMEM((B,), jnp.int32),
        pltpu.VMEM((B,), jnp.int32),
        pltpu.VMEM((B,), jnp.int32),
        pltpu.VMEM((B,), jnp.int32),
        pltpu.VMEM((B, D), jnp.float32),
        pltpu.VMEM((B, D), jnp.float32),
        pltpu.VMEM((B,), jnp.float32),
        pltpu.VMEM((B,), jnp.float32),
    ]
    f = functools.partial(
        pl.kernel,
        mesh=mesh,
        compiler_params=pltpu.CompilerParams(needs_layout_passes=False),
        out_type=jax.ShapeDtypeStruct((H * N, D), jnp.float32),
        scratch_types=(
            set_bufs + set_bufs + [
                pltpu.VMEM((EPT,), jnp.float32),
                pltpu.VMEM((B * 16,), jnp.float32),
                pltpu.VMEM((640,), jnp.float32),
                pltpu.VMEM((640,), jnp.float32),
                pltpu.VMEM((H * D,), jnp.float32),
                pltpu.VMEM_SHARED((N,), jnp.float32),
                pltpu.VMEM_SHARED((N, D), jnp.float32),
                pltpu.SemaphoreType.DMA,
                pltpu.SemaphoreType.DMA,
                pltpu.SemaphoreType.DMA,
                pltpu.SemaphoreType.DMA,
                pltpu.SemaphoreType.DMA,
                pltpu.SemaphoreType.DMA,
                pltpu.SemaphoreType.DMA,
                pltpu.SemaphoreType.DMA,
            ]
        ),
    )(_sc_body)
    return f(xl_t, xr_t, src_r, dst_r, att_f)


def _final_body(agg_ref, bias_ref, pw_ref, pb_ref, x_ref, o_ref):
    agg = agg_ref[...]
    acc = x_ref[...] + pb_ref[...]
    for h in range(H):
        a = jnp.maximum(agg[h] + bias_ref[...][h], 0.0)
        acc = acc + jnp.dot(a, pw_ref[...][h], preferred_element_type=jnp.float32)
    o_ref[...] = acc


def _final(agg_h, bias_h, pw_h, proj_b, x):
    return pl.pallas_call(
        _final_body,
        grid=(NB,),
        in_specs=[
            pl.BlockSpec((H, BR, D), lambda i: (0, i, 0)),
            pl.BlockSpec((H, D), lambda i: (0, 0)),
            pl.BlockSpec((H, D, D), lambda i: (0, 0, 0)),
            pl.BlockSpec((D,), lambda i: (0,)),
            pl.BlockSpec((BR, D), lambda i: (i, 0)),
        ],
        out_specs=pl.BlockSpec((BR, D), lambda i: (i, 0)),
        out_shape=jax.ShapeDtypeStruct((N, D), jnp.float32),
    )(agg_h, bias_h, pw_h, proj_b, x)


def kernel(x, edge_index, W_l, W_r, att, bias, proj_W, proj_b):
    wl_h = W_l.reshape(D, H, D).transpose(1, 0, 2)
    wr_h = W_r.reshape(D, H, D).transpose(1, 0, 2)
    xl_t, xr_t = _prep(x, wl_h, wr_h)

    agg = _sc_edges(xl_t.reshape(H * N, D), xr_t.reshape(H * N, D),
                    edge_index[0], edge_index[1], att.reshape(H * D))

    pw_h = proj_W.reshape(H, D, D)
    bias_h = bias.reshape(H, D)
    return _final(agg.reshape(H, N, D), bias_h, pw_h, proj_b, x)


# speedup vs baseline: 1.0627x; 1.0627x over previous
import functools

import jax
import jax.numpy as jnp
from jax import lax
from jax.experimental import pallas as pl
from jax.experimental.pallas import tpu as pltpu
from jax.experimental.pallas import tpu_sc as plsc

N = 10000
E = 320000
D = 128
H = 4
NEG = 0.2

NT = 16
EPT = E // NT
B = 32
NCH = EPT // B
NG = B // 16
NB = 10
BR = N // NB


def _prep_body(x_ref, wl_ref, wr_ref, xl_ref, xr_ref):
    x = x_ref[...]
    xl_ref[0] = jnp.dot(x, wl_ref[0], preferred_element_type=jnp.float32)
    xr_ref[0] = jnp.dot(x, wr_ref[0], preferred_element_type=jnp.float32)


def _prep(x, wl_h, wr_h):
    return pl.pallas_call(
        _prep_body,
        grid=(H, NB),
        in_specs=[
            pl.BlockSpec((BR, D), lambda h, i: (i, 0)),
            pl.BlockSpec((1, D, D), lambda h, i: (h, 0, 0)),
            pl.BlockSpec((1, D, D), lambda h, i: (h, 0, 0)),
        ],
        out_specs=[
            pl.BlockSpec((1, BR, D), lambda h, i: (h, i, 0)),
            pl.BlockSpec((1, BR, D), lambda h, i: (h, i, 0)),
        ],
        out_shape=[
            jax.ShapeDtypeStruct((H, N, D), jnp.float32),
            jax.ShapeDtypeStruct((H, N, D), jnp.float32),
        ],
    )(x, wl_h, wr_h)


def _sc_body(xl_hbm, xr_hbm, src_hbm, dst_hbm, att_hbm, out_hbm,
             srcb0, dstb0, sidx0, didx0, ul0, ur0, ivb0, w0,
             srcb1, dstb1, sidx1, didx1, ul1, ur1, ivb1, w1,
             ex_v, acc_v, dch_v, z640_v, att_v,
             denom_s, out_s,
             semi0a, semi0b, semr0a, semr0b, sems0,
             semi1a, semi1b, semr1a, semr1b, sems1):
    cid = lax.axis_index("c")
    sid = lax.axis_index("s")

    bufA = (srcb0, dstb0, sidx0, didx0, ul0, ur0, ivb0, w0,
            semi0a, semr0a, semi0b, semr0b, sems0)
    bufB = (srcb1, dstb1, sidx1, didx1, ul1, ur1, ivb1, w1,
            semi1a, semr1a, semi1b, semr1b, sems1)

    pltpu.sync_copy(att_hbm, att_v)

    zv = jnp.zeros((16,), jnp.float32)
    lane = lax.iota(jnp.int32, 16)
    laneb = lane * 16

    def _z640(i, c):
        z640_v[pl.ds(i * 16, 16)] = zv
        return c
    lax.fori_loop(0, 40, _z640, 0)

    ebase = sid * EPT

    def _fire_idx(j, s):
        off = ebase + j * B
        c1 = pltpu.async_copy(src_hbm.at[pl.ds(off, B)], s[0], s[8])
        c2 = pltpu.async_copy(dst_hbm.at[pl.ds(off, B)], s[1], s[10])
        return (c1, c2)

    def _wait(cs):
        for c in cs:
            c.wait()

    for hh in range(2):
        head = cid * 2 + hh
        base = head * N

        def _zul(r, c):
            for k in range(8):
                ul0[r, pl.ds(k * 16, 16)] = zv
            return c
        lax.fori_loop(0, B, _zul, 0)

        @pl.when(sid < 15)
        def _():
            pltpu.sync_copy(z640_v, denom_s.at[pl.ds(sid * 640, 640)])
            for k in range(20):
                pltpu.sync_copy(ul0, out_s.at[pl.ds(sid * 640 + k * B, B)])

        @pl.when(sid == 15)
        def _():
            pltpu.sync_copy(z640_v.at[pl.ds(0, 400)],
                            denom_s.at[pl.ds(9600, 400)])
            for k in range(12):
                pltpu.sync_copy(ul0, out_s.at[pl.ds(9600 + k * B, B)])
            pltpu.sync_copy(ul0.at[pl.ds(0, 16)], out_s.at[pl.ds(9984, 16)])
        plsc.subcore_barrier()

        att_c = [att_v[pl.ds(head * D + c * 16, 16)] for c in range(8)]

        def _p1_build(s):
            for k in range(NG):
                s[2][pl.ds(k * 16, 16)] = s[0][pl.ds(k * 16, 16)] + base
                s[3][pl.ds(k * 16, 16)] = s[1][pl.ds(k * 16, 16)] + base

        def _p1_fire_rows(s):
            c1 = pltpu.async_copy(xl_hbm.at[s[2]], s[4], s[9])
            c2 = pltpu.async_copy(xr_hbm.at[s[3]], s[5], s[11])
            return (c1, c2)

        def _p1_compute(j, s):
            ul, ur = s[4], s[5]
            toff = pl.multiple_of(j * B, B)

            @plsc.parallel_loop(0, B, unroll=4)
            def _edge(e):
                acc = zv
                for c3 in range(8):
                    t = ul[e, pl.ds(c3 * 16, 16)] + ur[e, pl.ds(c3 * 16, 16)]
                    t = jnp.maximum(t, NEG * t)
                    acc = acc + att_c[c3] * t
                acc_v[pl.ds(e * 16, 16)] = acc

            @plsc.parallel_loop(0, NG)
            def _red(k):
                tot = zv
                for jj in range(16):
                    tot = tot + plsc.load_gather(acc_v,
                                                 [laneb + (k * 256 + jj)])
                ex_v[pl.ds(toff + k * 16, 16)] = jnp.exp(tot)
            return pltpu.async_copy(ex_v.at[pl.ds(toff, B)],
                                    denom_s.at[s[1]], s[12], add=True)

        def _p1_body(i, c):
            a = 2 * i
            b = 2 * i + 1
            ia = _fire_idx(a, bufA)
            ib = _fire_idx(b, bufB)
            _wait(ia)
            _p1_build(bufA)
            ra = _p1_fire_rows(bufA)
            _wait(ib)
            _p1_build(bufB)
            rb = _p1_fire_rows(bufB)
            _wait(ra)
            sa = _p1_compute(a, bufA)
            _wait(rb)
            sb = _p1_compute(b, bufB)
            _wait((sa, sb))
            return c
        lax.fori_loop(0, NCH // 2, _p1_body, 0)
        it = _fire_idx(NCH - 1, bufA)
        _wait(it)
        _p1_build(bufA)
        rt = _p1_fire_rows(bufA)
        _wait(rt)
        st = _p1_compute(NCH - 1, bufA)
        st.wait()
        plsc.subcore_barrier()

        def _inv_stripe(n, r0):
            pltpu.sync_copy(denom_s.at[pl.ds(r0, n)], dch_v.at[pl.ds(0, n)])

            def _i(i, c):
                v = dch_v[pl.ds(i * 16, 16)]
                dch_v[pl.ds(i * 16, 16)] = 1.0 / (v + 1e-16)
                return c
            lax.fori_loop(0, n // 16, _i, 0)
            pltpu.sync_copy(dch_v.at[pl.ds(0, n)], denom_s.at[pl.ds(r0, n)])

        @pl.when(sid < 15)
        def _():
            _inv_stripe(640, sid * 640)

        @pl.when(sid == 15)
        def _():
            _inv_stripe(400, 9600)
        plsc.subcore_barrier()

        def _p2_build(s):
            for k in range(NG):
                s[2][pl.ds(k * 16, 16)] = s[0][pl.ds(k * 16, 16)] + base

        def _p2_fire_rows(s):
            c1 = pltpu.async_copy(xl_hbm.at[s[2]], s[4], s[9])
            c2 = pltpu.async_copy(denom_s.at[s[1]], s[6], s[11])
            return (c1, c2)

        def _p2_compute(j, s):
            ul, ivb, wv = s[4], s[6], s[7]
            toff = pl.multiple_of(j * B, B)
            for k in range(NG):
                wv[pl.ds(k * 16, 16)] = (ex_v[pl.ds(toff + k * 16, 16)]
                                         * ivb[pl.ds(k * 16, 16)])

            @plsc.parallel_loop(0, B, unroll=4)
            def _edge(e):
                w = plsc.load_gather(wv, [jnp.full((16,), e, jnp.int32)])
                for c2 in range(8):
                    ul[e, pl.ds(c2 * 16, 16)] = w * ul[e, pl.ds(c2 * 16, 16)]
            return pltpu.async_copy(ul, out_s.at[s[1]], s[12], add=True)

        def _p2_body(i, c):
            a = 2 * i
            b = 2 * i + 1
            ia = _fire_idx(a, bufA)
            ib = _fire_idx(b, bufB)
            _wait(ia)
            _p2_build(bufA)
            ra = _p2_fire_rows(bufA)
            _wait(ib)
            _p2_build(bufB)
            rb = _p2_fire_rows(bufB)
            _wait(ra)
            sa = _p2_compute(a, bufA)
            _wait(rb)
            sb = _p2_compute(b, bufB)
            _wait((sa, sb))
            return c
        lax.fori_loop(0, NCH // 2, _p2_body, 0)
        it = _fire_idx(NCH - 1, bufA)
        _wait(it)
        _p2_build(bufA)
        rt = _p2_fire_rows(bufA)
        _wait(rt)
        st = _p2_compute(NCH - 1, bufA)
        st.wait()
        plsc.subcore_barrier()

        @pl.when(sid < 15)
        def _():
            pltpu.sync_copy(out_s.at[pl.ds(sid * 640, 640)],
                            out_hbm.at[pl.ds(base + sid * 640, 640)])

        @pl.when(sid == 15)
        def _():
            pltpu.sync_copy(out_s.at[pl.ds(9600, 400)],
                            out_hbm.at[pl.ds(base + 9600, 400)])


def _sc_edges(xl_t, xr_t, src_r, dst_r, att_f):
    mesh = plsc.VectorSubcoreMesh(core_axis_name="c", subcore_axis_name="s")
    set_bufs = [
        pltpu.VMEM((B,), jnp.int32),
        pltpu.VMEM((B,), jnp.int32),
        pltpu.VMEM((B,), jnp.int32),
        pltpu.VMEM((B,), jnp.int32),
        pltpu.VMEM((B, D), jnp.float32),
        pltpu.VMEM((B, D), jnp.float32),
        pltpu.VMEM((B,), jnp.float32),
        pltpu.VMEM((B,), jnp.float32),
    ]
    f = functools.partial(
        pl.kernel,
        mesh=mesh,
        compiler_params=pltpu.CompilerParams(needs_layout_passes=False),
        out_type=jax.ShapeDtypeStruct((H * N, D), jnp.float32),
        scratch_types=(
            set_bufs + set_bufs + [
                pltpu.VMEM((EPT,), jnp.float32),
                pltpu.VMEM((B * 16,), jnp.float32),
                pltpu.VMEM((640,), jnp.float32),
                pltpu.VMEM((640,), jnp.float32),
                pltpu.VMEM((H * D,), jnp.float32),
                pltpu.VMEM_SHARED((N,), jnp.float32),
                pltpu.VMEM_SHARED((N, D), jnp.float32),
                pltpu.SemaphoreType.DMA,
                pltpu.SemaphoreType.DMA,
                pltpu.SemaphoreType.DMA,
                pltpu.SemaphoreType.DMA,
                pltpu.SemaphoreType.DMA,
                pltpu.SemaphoreType.DMA,
                pltpu.SemaphoreType.DMA,
                pltpu.SemaphoreType.DMA,
                pltpu.SemaphoreType.DMA,
                pltpu.SemaphoreType.DMA,
            ]
        ),
    )(_sc_body)
    return f(xl_t, xr_t, src_r, dst_r, att_f)


def _final_body(agg_ref, bias_ref, pw_ref, pb_ref, x_ref, o_ref):
    agg = agg_ref[...]
    acc = x_ref[...] + pb_ref[...]
    for h in range(H):
        a = jnp.maximum(agg[h] + bias_ref[...][h], 0.0)
        acc = acc + jnp.dot(a, pw_ref[...][h], preferred_element_type=jnp.float32)
    o_ref[...] = acc


def _final(agg_h, bias_h, pw_h, proj_b, x):
    return pl.pallas_call(
        _final_body,
        grid=(NB,),
        in_specs=[
            pl.BlockSpec((H, BR, D), lambda i: (0, i, 0)),
            pl.BlockSpec((H, D), lambda i: (0, 0)),
            pl.BlockSpec((H, D, D), lambda i: (0, 0, 0)),
            pl.BlockSpec((D,), lambda i: (0,)),
            pl.BlockSpec((BR, D), lambda i: (i, 0)),
        ],
        out_specs=pl.BlockSpec((BR, D), lambda i: (i, 0)),
        out_shape=jax.ShapeDtypeStruct((N, D), jnp.float32),
    )(agg_h, bias_h, pw_h, proj_b, x)


def kernel(x, edge_index, W_l, W_r, att, bias, proj_W, proj_b):
    wl_h = W_l.reshape(D, H, D).transpose(1, 0, 2)
    wr_h = W_r.reshape(D, H, D).transpose(1, 0, 2)
    xl_t, xr_t = _prep(x, wl_h, wr_h)

    agg = _sc_edges(xl_t.reshape(H * N, D), xr_t.reshape(H * N, D),
                    edge_index[0], edge_index[1], att.reshape(H * D))

    pw_h = proj_W.reshape(H, D, D)
    bias_h = bias.reshape(H, D)
    return _final(agg.reshape(H, N, D), bias_h, pw_h, proj_b, x)


# speedup vs baseline: 1.3275x; 1.2492x over previous
import functools

import jax
import jax.numpy as jnp
from jax import lax
from jax.experimental import pallas as pl
from jax.experimental.pallas import tpu as pltpu
from jax.experimental.pallas import tpu_sc as plsc

N = 10000
E = 320000
D = 128
H = 4
NEG = 0.2

NT = 16
EPT = E // NT
B = 80
NCH = EPT // B
HB = 48
NB = 10
BR = N // NB


def _prep_body(x_ref, wl_ref, wr_ref, xl_ref, xr_ref):
    x = x_ref[...]
    xl_ref[0] = jnp.dot(x, wl_ref[0], preferred_element_type=jnp.float32)
    xr_ref[0] = jnp.dot(x, wr_ref[0], preferred_element_type=jnp.float32)


def _prep(x, wl_h, wr_h):
    return pl.pallas_call(
        _prep_body,
        grid=(H, NB),
        in_specs=[
            pl.BlockSpec((BR, D), lambda h, i: (i, 0)),
            pl.BlockSpec((1, D, D), lambda h, i: (h, 0, 0)),
            pl.BlockSpec((1, D, D), lambda h, i: (h, 0, 0)),
        ],
        out_specs=[
            pl.BlockSpec((1, BR, D), lambda h, i: (h, i, 0)),
            pl.BlockSpec((1, BR, D), lambda h, i: (h, i, 0)),
        ],
        out_shape=[
            jax.ShapeDtypeStruct((H, N, D), jnp.float32),
            jax.ShapeDtypeStruct((H, N, D), jnp.float32),
        ],
    )(x, wl_h, wr_h)


def _sc_body(xl_hbm, xr_hbm, src_hbm, dst_hbm, att_hbm, out_hbm,
             srcb0, dstb0, sidx0, didx0, ivb0, w0,
             srcb1, dstb1, sidx1, didx1, ivb1, w1,
             ul_v, ur_v, ex_v, acc_v, dch_v, z640_v, att_v,
             denom_s, out_s,
             semi0a, semi0b, sems0,
             semi1a, semi1b, sems1,
             semL1, semL2, semR1, semR2):
    cid = lax.axis_index("c")
    sid = lax.axis_index("s")

    bufA = (srcb0, dstb0, sidx0, didx0, ivb0, w0, semi0a, semi0b, sems0)
    bufB = (srcb1, dstb1, sidx1, didx1, ivb1, w1, semi1a, semi1b, sems1)

    pltpu.sync_copy(att_hbm, att_v)

    zv = jnp.zeros((16,), jnp.float32)
    lane = lax.iota(jnp.int32, 16)
    laneb = lane * 16

    def _z640(i, c):
        z640_v[pl.ds(i * 16, 16)] = zv
        return c
    lax.fori_loop(0, 40, _z640, 0)

    ebase = sid * EPT

    def _fire_idx(j, s):
        off = ebase + j * B
        c1 = pltpu.async_copy(src_hbm.at[pl.ds(off, B)], s[0], s[6])
        c2 = pltpu.async_copy(dst_hbm.at[pl.ds(off, B)], s[1], s[7])
        return (c1, c2)

    def _wait(cs):
        for c in cs:
            c.wait()

    for hh in range(2):
        head = cid * 2 + hh
        base = head * N

        def _zul(r, c):
            for k in range(8):
                ul_v[r, pl.ds(k * 16, 16)] = zv
            return c
        lax.fori_loop(0, B, _zul, 0)

        @pl.when(sid < 15)
        def _():
            pltpu.sync_copy(z640_v, denom_s.at[pl.ds(sid * 640, 640)])
            for k in range(8):
                pltpu.sync_copy(ul_v, out_s.at[pl.ds(sid * 640 + k * B, B)])

        @pl.when(sid == 15)
        def _():
            pltpu.sync_copy(z640_v.at[pl.ds(0, 400)],
                            denom_s.at[pl.ds(9600, 400)])
            for k in range(5):
                pltpu.sync_copy(ul_v, out_s.at[pl.ds(9600 + k * B, B)])
        plsc.subcore_barrier()

        att_c = [att_v[pl.ds(head * D + c * 16, 16)] for c in range(8)]

        def _p1_build(s):
            for k in range(5):
                s[2][pl.ds(k * 16, 16)] = s[0][pl.ds(k * 16, 16)] + base
                s[3][pl.ds(k * 16, 16)] = s[1][pl.ds(k * 16, 16)] + base

        def _p1_fire_rows(s):
            h1 = (pltpu.async_copy(xl_hbm.at[s[2].at[pl.ds(0, HB)]],
                                   ul_v.at[pl.ds(0, HB)], semL1),
                  pltpu.async_copy(xr_hbm.at[s[3].at[pl.ds(0, HB)]],
                                   ur_v.at[pl.ds(0, HB)], semR1))
            h2 = (pltpu.async_copy(xl_hbm.at[s[2].at[pl.ds(HB, B - HB)]],
                                   ul_v.at[pl.ds(HB, B - HB)], semL2),
                  pltpu.async_copy(xr_hbm.at[s[3].at[pl.ds(HB, B - HB)]],
                                   ur_v.at[pl.ds(HB, B - HB)], semR2))
            return h1, h2

        def _p1_half(j, e0, e1):
            toff = pl.multiple_of(j * B, B)

            @plsc.parallel_loop(e0, e1, unroll=4)
            def _edge(e):
                acc = zv
                for c3 in range(8):
                    t = ul_v[e, pl.ds(c3 * 16, 16)] + ur_v[e, pl.ds(c3 * 16, 16)]
                    t = jnp.maximum(t, NEG * t)
                    acc = acc + att_c[c3] * t
                acc_v[pl.ds(e * 16, 16)] = acc

            @plsc.parallel_loop(e0 // 16, e1 // 16)
            def _red(k):
                tot = zv
                for jj in range(16):
                    tot = tot + plsc.load_gather(acc_v,
                                                 [laneb + (k * 256 + jj)])
                ex_v[pl.ds(toff + k * 16, 16)] = jnp.exp(tot)

        def _p1_chunk(j, s):
            h1, h2 = _p1_fire_rows(s)
            _wait(h1)
            _p1_half(j, 0, HB)
            _wait(h2)
            _p1_half(j, HB, B)
            toff = pl.multiple_of(j * B, B)
            return pltpu.async_copy(ex_v.at[pl.ds(toff, B)],
                                    denom_s.at[s[1]], s[8], add=True)

        def _p1_body(i, c):
            a = 2 * i
            b = 2 * i + 1
            ia = _fire_idx(a, bufA)
            ib = _fire_idx(b, bufB)
            _wait(ia)
            _p1_build(bufA)
            sa = _p1_chunk(a, bufA)
            _wait(ib)
            _p1_build(bufB)
            sb = _p1_chunk(b, bufB)
            _wait((sa, sb))
            return c
        lax.fori_loop(0, NCH // 2, _p1_body, 0)
        plsc.subcore_barrier()

        def _inv_stripe(n, r0):
            pltpu.sync_copy(denom_s.at[pl.ds(r0, n)], dch_v.at[pl.ds(0, n)])

            def _i(i, c):
                v = dch_v[pl.ds(i * 16, 16)]
                dch_v[pl.ds(i * 16, 16)] = 1.0 / (v + 1e-16)
                return c
            lax.fori_loop(0, n // 16, _i, 0)
            pltpu.sync_copy(dch_v.at[pl.ds(0, n)], denom_s.at[pl.ds(r0, n)])

        @pl.when(sid < 15)
        def _():
            _inv_stripe(640, sid * 640)

        @pl.when(sid == 15)
        def _():
            _inv_stripe(400, 9600)
        plsc.subcore_barrier()

        def _p2_build(s):
            for k in range(5):
                s[2][pl.ds(k * 16, 16)] = s[0][pl.ds(k * 16, 16)] + base

        def _p2_fire_rows(s, rows, semrow, semiv):
            c1 = pltpu.async_copy(xl_hbm.at[s[2]], rows, semrow)
            c2 = pltpu.async_copy(denom_s.at[s[1]], s[4], semiv)
            return (c1, c2)

        def _p2_compute(j, s, rows):
            toff = pl.multiple_of(j * B, B)
            for k in range(5):
                s[5][pl.ds(k * 16, 16)] = (ex_v[pl.ds(toff + k * 16, 16)]
                                           * s[4][pl.ds(k * 16, 16)])
            wv = s[5]

            @plsc.parallel_loop(0, B, unroll=4)
            def _edge(e):
                w = plsc.load_gather(wv, [jnp.full((16,), e, jnp.int32)])
                for c2 in range(8):
                    rows[e, pl.ds(c2 * 16, 16)] = w * rows[e, pl.ds(c2 * 16, 16)]
            return pltpu.async_copy(rows, out_s.at[s[1]], s[8], add=True)

        def _p2_body(i, c):
            a = 2 * i
            b = 2 * i + 1
            ia = _fire_idx(a, bufA)
            ib = _fire_idx(b, bufB)
            _wait(ia)
            _p2_build(bufA)
            ra = _p2_fire_rows(bufA, ul_v, semL1, semR1)
            _wait(ib)
            _p2_build(bufB)
            rb = _p2_fire_rows(bufB, ur_v, semL2, semR2)
            _wait(ra)
            sa = _p2_compute(a, bufA, ul_v)
            _wait(rb)
            sb = _p2_compute(b, bufB, ur_v)
            _wait((sa, sb))
            return c
        lax.fori_loop(0, NCH // 2, _p2_body, 0)
        plsc.subcore_barrier()

        @pl.when(sid < 15)
        def _():
            pltpu.sync_copy(out_s.at[pl.ds(sid * 640, 640)],
                            out_hbm.at[pl.ds(base + sid * 640, 640)])

        @pl.when(sid == 15)
        def _():
            pltpu.sync_copy(out_s.at[pl.ds(9600, 400)],
                            out_hbm.at[pl.ds(base + 9600, 400)])


def _sc_edges(xl_t, xr_t, src_r, dst_r, att_f):
    mesh = plsc.VectorSubcoreMesh(core_axis_name="c", subcore_axis_name="s")
    set_bufs = [
        pltpu.VMEM((B,), jnp.int32),
        pltpu.VMEM((B,), jnp.int32),
        pltpu.VMEM((B,), jnp.int32),
        pltpu.VMEM((B,), jnp.int32),
        pltpu.VMEM((B,), jnp.float32),
        pltpu.VMEM((B,), jnp.float32),
    ]
    f = functools.partial(
        pl.kernel,
        mesh=mesh,
        compiler_params=pltpu.CompilerParams(needs_layout_passes=False),
        out_type=jax.ShapeDtypeStruct((H * N, D), jnp.float32),
        scratch_types=(
            set_bufs + set_bufs + [
                pltpu.VMEM((B, D), jnp.float32),
                pltpu.VMEM((B, D), jnp.float32),
                pltpu.VMEM((EPT,), jnp.float32),
                pltpu.VMEM((B * 16,), jnp.float32),
                pltpu.VMEM((640,), jnp.float32),
                pltpu.VMEM((640,), jnp.float32),
                pltpu.VMEM((H * D,), jnp.float32),
                pltpu.VMEM_SHARED((N,), jnp.float32),
                pltpu.VMEM_SHARED((N, D), jnp.float32),
            ] + [pltpu.SemaphoreType.DMA] * 10
        ),
    )(_sc_body)
    return f(xl_t, xr_t, src_r, dst_r, att_f)


def _final_body(agg_ref, bias_ref, pw_ref, pb_ref, x_ref, o_ref):
    agg = agg_ref[...]
    acc = x_ref[...] + pb_ref[...]
    for h in range(H):
        a = jnp.maximum(agg[h] + bias_ref[...][h], 0.0)
        acc = acc + jnp.dot(a, pw_ref[...][h], preferred_element_type=jnp.float32)
    o_ref[...] = acc


def _final(agg_h, bias_h, pw_h, proj_b, x):
    return pl.pallas_call(
        _final_body,
        grid=(NB,),
        in_specs=[
            pl.BlockSpec((H, BR, D), lambda i: (0, i, 0)),
            pl.BlockSpec((H, D), lambda i: (0, 0)),
            pl.BlockSpec((H, D, D), lambda i: (0, 0, 0)),
            pl.BlockSpec((D,), lambda i: (0,)),
            pl.BlockSpec((BR, D), lambda i: (i, 0)),
        ],
        out_specs=pl.BlockSpec((BR, D), lambda i: (i, 0)),
        out_shape=jax.ShapeDtypeStruct((N, D), jnp.float32),
    )(agg_h, bias_h, pw_h, proj_b, x)


def kernel(x, edge_index, W_l, W_r, att, bias, proj_W, proj_b):
    wl_h = W_l.reshape(D, H, D).transpose(1, 0, 2)
    wr_h = W_r.reshape(D, H, D).transpose(1, 0, 2)
    xl_t, xr_t = _prep(x, wl_h, wr_h)

    agg = _sc_edges(xl_t.reshape(H * N, D), xr_t.reshape(H * N, D),
                    edge_index[0], edge_index[1], att.reshape(H * D))

    pw_h = proj_W.reshape(H, D, D)
    bias_h = bias.reshape(H, D)
    return _final(agg.reshape(H, N, D), bias_h, pw_h, proj_b, x)
